# baseline (device time: 31698 ns/iter reference)
import jax
import jax.numpy as jnp
from jax import lax
from jax.experimental import pallas as pl
from jax.experimental.pallas import tpu as pltpu

N_DEV = 4
NSUB = 2


def kernel(x, W1, W2):
    m, d = x.shape
    f = W1.shape[1]
    M = N_DEV * m
    q = m // 2
    s = q // NSUB

    def body(x_ref, w1_ref, w2_ref, out_ref,
             agcR_ref, agcL_ref, w1f_ref, w2f_ref, w1b_ref, w2b_ref, acc_ref,
             rscR_ref, rscL_ref, w_sems,
             agR_send, agR_recv, agL_send, agL_recv,
             rsR_send, rsR_recv, rsL_send, rsL_recv):
        my = lax.axis_index("i")
        left = (my - 1) % N_DEV
        right = (my + 1) % N_DEV

        w1_copy = pltpu.make_async_copy(w1_ref, w1f_ref, w_sems.at[0])
        w2_copy = pltpu.make_async_copy(w2_ref, w2f_ref, w_sems.at[1])
        w1_copy.start()
        w2_copy.start()

        def copy(src, dst, ssem, rsem, target):
            return pltpu.make_async_remote_copy(
                src_ref=src, dst_ref=dst, send_sem=ssem, recv_sem=rsem,
                device_id=(target,), device_id_type=pl.DeviceIdType.MESH,
            )

        def compute_sub(origin, src_slot, is_b, j):
            hblk = jnp.dot(src_slot[:, :], w1b_ref[:, :],
                           preferred_element_type=jnp.float32
                           ).astype(jnp.bfloat16)
            one = jnp.bfloat16(1.0)
            hblk = hblk * (one / (one + jnp.exp(-hblk)))
            row0 = origin * m + (q if is_b else 0) + j * s
            acc_ref[pl.ds(row0, s), :] = jnp.dot(
                hblk, w2b_ref[:, :],
                preferred_element_type=jnp.float32)

        def acc_sub(b, is_b, j):
            return acc_ref[pl.ds(b * m + (q if is_b else 0) + j * s, s), :]

        barrier_sem = pltpu.get_barrier_semaphore()
        for nbr in (left, right):
            pl.semaphore_signal(
                barrier_sem, inc=1,
                device_id=(nbr,), device_id_type=pl.DeviceIdType.MESH,
            )
        pl.semaphore_wait(barrier_sem, 2)

        agcR_ref[0, :, :] = x_ref[pl.ds(0, q), :].astype(jnp.bfloat16)
        agcL_ref[0, :, :] = x_ref[pl.ds(q, q), :].astype(jnp.bfloat16)
        agR = [[None] * NSUB for _ in range(N_DEV - 1)]
        agL = [[None] * NSUB for _ in range(N_DEV - 1)]
        rsR = [[None] * NSUB for _ in range(N_DEV - 1)]
        rsL = [[None] * NSUB for _ in range(N_DEV - 1)]
        for j in range(NSUB):
            agR[0][j] = copy(agcR_ref.at[0, pl.ds(j * s, s)],
                             agcR_ref.at[1, pl.ds(j * s, s)],
                             agR_send.at[0, j], agR_recv.at[0, j], right)
            agR[0][j].start()
            agL[0][j] = copy(agcL_ref.at[0, pl.ds(j * s, s)],
                             agcL_ref.at[1, pl.ds(j * s, s)],
                             agL_send.at[0, j], agL_recv.at[0, j], left)
            agL[0][j].start()

        w1_copy.wait()
        w1b_ref[:, :] = w1f_ref[:, :].astype(jnp.bfloat16)
        w2_copy.wait()
        w2b_ref[:, :] = w2f_ref[:, :].astype(jnp.bfloat16)
        for j in range(NSUB):
            compute_sub(my, agcR_ref.at[0, pl.ds(j * s, s)], False, j)
            compute_sub(my, agcL_ref.at[0, pl.ds(j * s, s)], True, j)

        def ring_step(h, j, ag, agc_ref, ag_send, ag_recv,
                      rs, rsc_ref, rs_send, rs_recv, origin, is_b, target):
            sub = pl.ds(j * s, s)
            ag[h][j].wait_recv()
            if h < N_DEV - 2:
                ag[h + 1][j] = copy(agc_ref.at[h + 1, sub],
                                    agc_ref.at[h + 2, sub],
                                    ag_send.at[h + 1, j],
                                    ag_recv.at[h + 1, j], target)
                ag[h + 1][j].start()
            compute_sub(origin, agc_ref.at[h + 1, sub], is_b, j)
            if h == 0:
                rsc_ref[0, sub] = acc_sub(origin, is_b, j).astype(jnp.bfloat16)
            else:
                rs[h - 1][j].wait_recv()
                rsc_ref[h, sub] = (
                    rsc_ref[h, sub].astype(jnp.float32)
                    + acc_sub(origin, is_b, j)
                ).astype(jnp.bfloat16)
            rs[h][j] = copy(rsc_ref.at[h, sub], rsc_ref.at[h + 1, sub],
                            rs_send.at[h, j], rs_recv.at[h, j], target)
            rs[h][j].start()

        for h in range(N_DEV - 1):
            oR = (my - h - 1) % N_DEV
            oL = (my + h + 1) % N_DEV
            for j in range(NSUB):
                stepR = (ring_step, (h, j, agR, agcR_ref, agR_send, agR_recv,
                                     rsR, rscR_ref, rsR_send, rsR_recv,
                                     oR, False, right))
                stepL = (ring_step, (h, j, agL, agcL_ref, agL_send, agL_recv,
                                     rsL, rscL_ref, rsL_send, rsL_recv,
                                     oL, True, left))
                first, second = (stepR, stepL) if j % 2 == 0 else (stepL, stepR)
                first[0](*first[1])
                second[0](*second[1])

        for j in range(NSUB):
            sub = pl.ds(j * s, s)
            rsR[N_DEV - 2][j].wait_recv()
            out_ref[pl.ds(j * s, s), :] = (
                rscR_ref[N_DEV - 1, sub].astype(jnp.float32)
                + acc_sub(my, False, j)
            )
            rsL[N_DEV - 2][j].wait_recv()
            out_ref[pl.ds(q + j * s, s), :] = (
                rscL_ref[N_DEV - 1, sub].astype(jnp.float32)
                + acc_sub(my, True, j)
            )

        for h in range(N_DEV - 1):
            for j in range(NSUB):
                agR[h][j].wait_send()
                agL[h][j].wait_send()
                rsR[h][j].wait_send()
                rsL[h][j].wait_send()

    return pl.pallas_call(
        body,
        out_shape=jax.ShapeDtypeStruct((m, d), jnp.float32),
        in_specs=[
            pl.BlockSpec(memory_space=pltpu.VMEM),
            pl.BlockSpec(memory_space=pltpu.MemorySpace.HBM),
            pl.BlockSpec(memory_space=pltpu.MemorySpace.HBM),
        ],
        out_specs=pl.BlockSpec(memory_space=pltpu.VMEM),
        scratch_shapes=[
            pltpu.VMEM((N_DEV, q, d), jnp.bfloat16),
            pltpu.VMEM((N_DEV, q, d), jnp.bfloat16),
            pltpu.VMEM((d, f), jnp.float32),
            pltpu.VMEM((f, d), jnp.float32),
            pltpu.VMEM((d, f), jnp.bfloat16),
            pltpu.VMEM((f, d), jnp.bfloat16),
            pltpu.VMEM((M, d), jnp.float32),
            pltpu.VMEM((N_DEV, q, d), jnp.bfloat16),
            pltpu.VMEM((N_DEV, q, d), jnp.bfloat16),
            pltpu.SemaphoreType.DMA((2,)),
            pltpu.SemaphoreType.DMA((N_DEV - 1, NSUB)),
            pltpu.SemaphoreType.DMA((N_DEV - 1, NSUB)),
            pltpu.SemaphoreType.DMA((N_DEV - 1, NSUB)),
            pltpu.SemaphoreType.DMA((N_DEV - 1, NSUB)),
            pltpu.SemaphoreType.DMA((N_DEV - 1, NSUB)),
            pltpu.SemaphoreType.DMA((N_DEV - 1, NSUB)),
            pltpu.SemaphoreType.DMA((N_DEV - 1, NSUB)),
            pltpu.SemaphoreType.DMA((N_DEV - 1, NSUB)),
        ],
        compiler_params=pltpu.CompilerParams(collective_id=0),
    )(x, W1, W2)


# device time: 30800 ns/iter; 1.0292x vs baseline; 1.0292x over previous
import jax
import jax.numpy as jnp
from jax import lax
from jax.experimental import pallas as pl
from jax.experimental.pallas import tpu as pltpu

N_DEV = 4
NSUB = 2


def kernel(x, W1, W2):
    m, d = x.shape
    f = W1.shape[1]
    M = N_DEV * m
    q = m // 2
    s = q // NSUB

    def body(x_ref, w1_ref, w2_ref, out_ref,
             agcR_ref, agcL_ref, w1f_ref, w2f_ref, w1b_ref, w2b_ref, acc_ref,
             rscR_ref, rscL_ref, w_sems,
             agR_send, agR_recv, agL_send, agL_recv,
             rsR_send, rsR_recv, rsL_send, rsL_recv):
        my = lax.axis_index("i")
        left = (my - 1) % N_DEV
        right = (my + 1) % N_DEV

        w1_copy = pltpu.make_async_copy(w1_ref, w1f_ref, w_sems.at[0])
        w2_copy = pltpu.make_async_copy(w2_ref, w2f_ref, w_sems.at[1])
        w1_copy.start()
        w2_copy.start()

        def copy(src, dst, ssem, rsem, target):
            return pltpu.make_async_remote_copy(
                src_ref=src, dst_ref=dst, send_sem=ssem, recv_sem=rsem,
                device_id=(target,), device_id_type=pl.DeviceIdType.MESH,
            )

        def compute_sub(origin, src_slot, is_b, j):
            hblk = jnp.dot(src_slot[:, :], w1b_ref[:, :],
                           preferred_element_type=jnp.float32
                           ).astype(jnp.bfloat16)
            one = jnp.bfloat16(1.0)
            hblk = hblk * (one / (one + jnp.exp(-hblk)))
            row0 = origin * m + (q if is_b else 0) + j * s
            acc_ref[pl.ds(row0, s), :] = jnp.dot(
                hblk, w2b_ref[:, :],
                preferred_element_type=jnp.float32)

        def acc_sub(b, is_b, j):
            return acc_ref[pl.ds(b * m + (q if is_b else 0) + j * s, s), :]

        barrier_sem = pltpu.get_barrier_semaphore()
        for nbr in (left, right):
            pl.semaphore_signal(
                barrier_sem, inc=1,
                device_id=(nbr,), device_id_type=pl.DeviceIdType.MESH,
            )
        pl.semaphore_wait(barrier_sem, 2)

        agcR_ref[0, :, :] = x_ref[pl.ds(0, q), :].astype(jnp.bfloat16)
        agcL_ref[0, :, :] = x_ref[pl.ds(q, q), :].astype(jnp.bfloat16)
        agR = [[None] * NSUB for _ in range(N_DEV - 1)]
        agL = [[None] * NSUB for _ in range(N_DEV - 1)]
        rsR = [[None] * NSUB for _ in range(N_DEV - 1)]
        rsL = [[None] * NSUB for _ in range(N_DEV - 1)]
        for j in range(NSUB):
            agR[0][j] = copy(agcR_ref.at[0, pl.ds(j * s, s)],
                             agcR_ref.at[1, pl.ds(j * s, s)],
                             agR_send.at[0, j], agR_recv.at[0, j], right)
            agR[0][j].start()
            agL[0][j] = copy(agcL_ref.at[0, pl.ds(j * s, s)],
                             agcL_ref.at[1, pl.ds(j * s, s)],
                             agL_send.at[0, j], agL_recv.at[0, j], left)
            agL[0][j].start()

        w1_copy.wait()
        w1b_ref[:, :] = w1f_ref[:, :].astype(jnp.bfloat16)
        w2_copy.wait()
        w2b_ref[:, :] = w2f_ref[:, :].astype(jnp.bfloat16)
        for j in range(NSUB):
            compute_sub(my, agcR_ref.at[0, pl.ds(j * s, s)], False, j)
            compute_sub(my, agcL_ref.at[0, pl.ds(j * s, s)], True, j)

        def ring_step(h, j, ag, agc_ref, ag_send, ag_recv,
                      rs, rsc_ref, rs_send, rs_recv, origin, is_b, target):
            sub = pl.ds(j * s, s)
            ag[h][j].wait_recv()
            if h < N_DEV - 2:
                ag[h + 1][j] = copy(agc_ref.at[h + 1, sub],
                                    agc_ref.at[h + 2, sub],
                                    ag_send.at[h + 1, j],
                                    ag_recv.at[h + 1, j], target)
                ag[h + 1][j].start()
            compute_sub(origin, agc_ref.at[h + 1, sub], is_b, j)
            if h == 0:
                rsc_ref[0, sub] = acc_sub(origin, is_b, j).astype(jnp.bfloat16)
            else:
                rs[h - 1][j].wait_recv()
                rsc_ref[h, sub] = (
                    rsc_ref[h, sub].astype(jnp.float32)
                    + acc_sub(origin, is_b, j)
                ).astype(jnp.bfloat16)
            rs[h][j] = copy(rsc_ref.at[h, sub], rsc_ref.at[h + 1, sub],
                            rs_send.at[h, j], rs_recv.at[h, j], target)
            rs[h][j].start()

        for h in range(N_DEV - 1):
            oR = (my - h - 1) % N_DEV
            oL = (my + h + 1) % N_DEV
            for j in range(NSUB):
                ring_step(h, j, agR, agcR_ref, agR_send, agR_recv,
                          rsR, rscR_ref, rsR_send, rsR_recv, oR, False, right)
                ring_step(h, j, agL, agcL_ref, agL_send, agL_recv,
                          rsL, rscL_ref, rsL_send, rsL_recv, oL, True, left)

        for j in range(NSUB):
            sub = pl.ds(j * s, s)
            rsR[N_DEV - 2][j].wait_recv()
            out_ref[pl.ds(j * s, s), :] = (
                rscR_ref[N_DEV - 1, sub].astype(jnp.float32)
                + acc_sub(my, False, j)
            )
            rsL[N_DEV - 2][j].wait_recv()
            out_ref[pl.ds(q + j * s, s), :] = (
                rscL_ref[N_DEV - 1, sub].astype(jnp.float32)
                + acc_sub(my, True, j)
            )

        for h in range(N_DEV - 1):
            for j in range(NSUB):
                agR[h][j].wait_send()
                agL[h][j].wait_send()
                rsR[h][j].wait_send()
                rsL[h][j].wait_send()

    return pl.pallas_call(
        body,
        out_shape=jax.ShapeDtypeStruct((m, d), jnp.float32),
        in_specs=[
            pl.BlockSpec(memory_space=pltpu.VMEM),
            pl.BlockSpec(memory_space=pltpu.MemorySpace.HBM),
            pl.BlockSpec(memory_space=pltpu.MemorySpace.HBM),
        ],
        out_specs=pl.BlockSpec(memory_space=pltpu.VMEM),
        scratch_shapes=[
            pltpu.VMEM((N_DEV, q, d), jnp.bfloat16),
            pltpu.VMEM((N_DEV, q, d), jnp.bfloat16),
            pltpu.VMEM((d, f), jnp.float32),
            pltpu.VMEM((f, d), jnp.float32),
            pltpu.VMEM((d, f), jnp.bfloat16),
            pltpu.VMEM((f, d), jnp.bfloat16),
            pltpu.VMEM((M, d), jnp.float32),
            pltpu.VMEM((N_DEV, q, d), jnp.bfloat16),
            pltpu.VMEM((N_DEV, q, d), jnp.bfloat16),
            pltpu.SemaphoreType.DMA((2,)),
            pltpu.SemaphoreType.DMA((N_DEV - 1, NSUB)),
            pltpu.SemaphoreType.DMA((N_DEV - 1, NSUB)),
            pltpu.SemaphoreType.DMA((N_DEV - 1, NSUB)),
            pltpu.SemaphoreType.DMA((N_DEV - 1, NSUB)),
            pltpu.SemaphoreType.DMA((N_DEV - 1, NSUB)),
            pltpu.SemaphoreType.DMA((N_DEV - 1, NSUB)),
            pltpu.SemaphoreType.DMA((N_DEV - 1, NSUB)),
            pltpu.SemaphoreType.DMA((N_DEV - 1, NSUB)),
        ],
        compiler_params=pltpu.CompilerParams(collective_id=0),
    )(x, W1, W2)
